# baseline (speedup 1.0000x reference)
import jax
import jax.numpy as jnp
from jax.experimental import pallas as pl
from jax.experimental.pallas import tpu as pltpu

N = 4096
H = 256
BM = 512
NB = N // BM
BK = 2048
KB = N // BK


def _gcn_kernel(x_ref, Ap_ref, An_ref, WinT_ref, bin_ref,
                Wp0T_ref, Wn0T_ref, b0_ref,
                Wp1T_ref, Wn1T_ref, b1_ref,
                out_ref, h0_ref, h1_ref, hp_ref, hn_ref):
    s = pl.program_id(0)
    k = pl.program_id(1)

    @pl.when((s == 0) & (k == 0))
    def _prep():
        h0_ref[...] = jnp.tanh(
            jnp.dot(x_ref[...], WinT_ref[...],
                    preferred_element_type=jnp.float32)
            + bin_ref[...]
        )

    def accum(h_ref):
        hs = h_ref[pl.ds(k * BK, BK), :]
        hp = jnp.dot(Ap_ref[...], hs, preferred_element_type=jnp.float32)
        hn = jnp.dot(An_ref[...], hs, preferred_element_type=jnp.float32)

        @pl.when(k == 0)
        def _init():
            hp_ref[...] = hp
            hn_ref[...] = hn

        @pl.when(k != 0)
        def _acc():
            hp_ref[...] += hp
            hn_ref[...] += hn

    def epilogue(WpT, WnT, b):
        return jnp.tanh(
            jnp.dot(hp_ref[...], WpT, preferred_element_type=jnp.float32)
            + jnp.dot(hn_ref[...], WnT, preferred_element_type=jnp.float32)
            + b
        )

    @pl.when(s < NB)
    def _layer0():
        accum(h0_ref)

        @pl.when(k == KB - 1)
        def _fin0():
            h1_ref[pl.ds(s * BM, BM), :] = epilogue(
                Wp0T_ref[...], Wn0T_ref[...], b0_ref[...])

    @pl.when(s >= NB)
    def _layer1():
        accum(h1_ref)

        @pl.when(k == KB - 1)
        def _fin1():
            out_ref[...] = epilogue(
                Wp1T_ref[...], Wn1T_ref[...], b1_ref[...])


def _tile_spec():
    return pl.BlockSpec((BM, BK), lambda s, k: (s % NB, k))


def _full_spec(shape):
    return pl.BlockSpec(shape, lambda s, k: (0,) * len(shape))


@jax.jit
def kernel(x, A_pos, A_neg, W_in, b_in, W_pos0, b_pos0, W_neg0, b_neg0,
           W_pos1, b_pos1, W_neg1, b_neg1):
    f32 = jnp.float32
    return pl.pallas_call(
        _gcn_kernel,
        grid=(2 * NB, KB),
        in_specs=[
            _full_spec((N, H)),
            _tile_spec(),
            _tile_spec(),
            _full_spec((H, H)),
            _full_spec((1, H)),
            _full_spec((H, H)),
            _full_spec((H, H)),
            _full_spec((1, H)),
            _full_spec((H, H)),
            _full_spec((H, H)),
            _full_spec((1, H)),
        ],
        out_specs=pl.BlockSpec((BM, H),
                               lambda s, k: (jnp.maximum(s - NB, 0), 0)),
        out_shape=jax.ShapeDtypeStruct((N, H), f32),
        scratch_shapes=[
            pltpu.VMEM((N, H), f32),
            pltpu.VMEM((N, H), f32),
            pltpu.VMEM((BM, H), f32),
            pltpu.VMEM((BM, H), f32),
        ],
    )(x, A_pos, A_neg, W_in.T, b_in.reshape(1, H),
      W_pos0.T, W_neg0.T, (b_pos0 + b_neg0).reshape(1, H),
      W_pos1.T, W_neg1.T, (b_pos1 + b_neg1).reshape(1, H))


# speedup vs baseline: 1.1096x; 1.1096x over previous
import jax
import jax.numpy as jnp
from jax.experimental import pallas as pl
from jax.experimental.pallas import tpu as pltpu

N = 4096
H = 256
BM = 512
NB = N // BM
BK = 2048
KB = N // BK

_DNT = (((1,), (1,)), ((), ()))


def _gcn_kernel(x_ref, Ap_ref, An_ref, Win_ref, bin_ref,
                Wp0_ref, bp0_ref, Wn0_ref, bn0_ref,
                Wp1_ref, bp1_ref, Wn1_ref, bn1_ref,
                out_ref, h0_ref, h1_ref, hp_ref, hn_ref):
    s = pl.program_id(0)
    k = pl.program_id(1)

    @pl.when((s == 0) & (k == 0))
    def _prep():
        h0_ref[...] = jnp.tanh(
            jax.lax.dot_general(x_ref[...], Win_ref[...], _DNT,
                                preferred_element_type=jnp.float32)
            + bin_ref[...]
        )

    def accum(h_ref):
        hs = h_ref[pl.ds(k * BK, BK), :]
        hp = jnp.dot(Ap_ref[...], hs, preferred_element_type=jnp.float32)
        hn = jnp.dot(An_ref[...], hs, preferred_element_type=jnp.float32)

        @pl.when(k == 0)
        def _init():
            hp_ref[...] = hp
            hn_ref[...] = hn

        @pl.when(k != 0)
        def _acc():
            hp_ref[...] += hp
            hn_ref[...] += hn

    def epilogue(Wp, bp, Wn, bn):
        return jnp.tanh(
            jax.lax.dot_general(hp_ref[...], Wp, _DNT,
                                preferred_element_type=jnp.float32)
            + bp
            + jax.lax.dot_general(hn_ref[...], Wn, _DNT,
                                  preferred_element_type=jnp.float32)
            + bn
        )

    @pl.when(s < NB)
    def _layer0():
        accum(h0_ref)

        @pl.when(k == KB - 1)
        def _fin0():
            h1_ref[pl.ds(s * BM, BM), :] = epilogue(
                Wp0_ref[...], bp0_ref[...], Wn0_ref[...], bn0_ref[...])

    @pl.when(s >= NB)
    def _layer1():
        accum(h1_ref)

        @pl.when(k == KB - 1)
        def _fin1():
            out_ref[...] = epilogue(
                Wp1_ref[...], bp1_ref[...], Wn1_ref[...], bn1_ref[...])


def _tile_spec():
    return pl.BlockSpec((BM, BK), lambda s, k: (s % NB, k))


def _full_spec(shape):
    return pl.BlockSpec(shape, lambda s, k: (0,) * len(shape))


@jax.jit
def kernel(x, A_pos, A_neg, W_in, b_in, W_pos0, b_pos0, W_neg0, b_neg0,
           W_pos1, b_pos1, W_neg1, b_neg1):
    f32 = jnp.float32
    return pl.pallas_call(
        _gcn_kernel,
        grid=(2 * NB, KB),
        in_specs=[
            _full_spec((N, H)),
            _tile_spec(),
            _tile_spec(),
            _full_spec((H, H)),
            _full_spec((1, H)),
            _full_spec((H, H)),
            _full_spec((1, H)),
            _full_spec((H, H)),
            _full_spec((1, H)),
            _full_spec((H, H)),
            _full_spec((1, H)),
            _full_spec((H, H)),
            _full_spec((1, H)),
        ],
        out_specs=pl.BlockSpec((BM, H),
                               lambda s, k: (jnp.maximum(s - NB, 0), 0)),
        out_shape=jax.ShapeDtypeStruct((N, H), f32),
        scratch_shapes=[
            pltpu.VMEM((N, H), f32),
            pltpu.VMEM((N, H), f32),
            pltpu.VMEM((BM, H), f32),
            pltpu.VMEM((BM, H), f32),
        ],
    )(x, A_pos, A_neg, W_in, b_in.reshape(1, H),
      W_pos0, b_pos0.reshape(1, H), W_neg0, b_neg0.reshape(1, H),
      W_pos1, b_pos1.reshape(1, H), W_neg1, b_neg1.reshape(1, H))


# speedup vs baseline: 1.1370x; 1.0248x over previous
import jax
import jax.numpy as jnp
from jax.experimental import pallas as pl
from jax.experimental.pallas import tpu as pltpu

N = 4096
H = 256
BM = 512
NB = N // BM

_DNT = (((1,), (1,)), ((), ()))


def _gcn_kernel(x_ref, Ap_ref, An_ref, Win_ref, bin_ref,
                Wp0_ref, bp0_ref, Wn0_ref, bn0_ref,
                Wp1_ref, bp1_ref, Wn1_ref, bn1_ref,
                out_ref, h0_ref, h1_ref):
    s = pl.program_id(0)

    @pl.when(s == 0)
    def _prep():
        h0_ref[...] = jnp.tanh(
            jax.lax.dot_general(x_ref[...], Win_ref[...], _DNT,
                                preferred_element_type=jnp.float32)
            + bin_ref[...]
        )

    def stripe(h, Wp, bp, Wn, bn):
        hp = jnp.dot(Ap_ref[...], h, preferred_element_type=jnp.float32)
        hn = jnp.dot(An_ref[...], h, preferred_element_type=jnp.float32)
        return jnp.tanh(
            jax.lax.dot_general(hp, Wp, _DNT,
                                preferred_element_type=jnp.float32)
            + bp
            + jax.lax.dot_general(hn, Wn, _DNT,
                                  preferred_element_type=jnp.float32)
            + bn
        )

    @pl.when(s < NB)
    def _layer0():
        t = stripe(h0_ref[...], Wp0_ref[...], bp0_ref[...],
                   Wn0_ref[...], bn0_ref[...])
        h1_ref[pl.ds(s * BM, BM), :] = t

    @pl.when(s >= NB)
    def _layer1():
        out_ref[...] = stripe(h1_ref[...], Wp1_ref[...], bp1_ref[...],
                              Wn1_ref[...], bn1_ref[...])


def _stripe_spec():
    return pl.BlockSpec((BM, N), lambda s: (s % NB, 0))


def _full_spec(shape):
    return pl.BlockSpec(shape, lambda s: (0,) * len(shape))


@jax.jit
def kernel(x, A_pos, A_neg, W_in, b_in, W_pos0, b_pos0, W_neg0, b_neg0,
           W_pos1, b_pos1, W_neg1, b_neg1):
    f32 = jnp.float32
    return pl.pallas_call(
        _gcn_kernel,
        grid=(2 * NB,),
        in_specs=[
            _full_spec((N, H)),
            _stripe_spec(),
            _stripe_spec(),
            _full_spec((H, H)),
            _full_spec((1, H)),
            _full_spec((H, H)),
            _full_spec((1, H)),
            _full_spec((H, H)),
            _full_spec((1, H)),
            _full_spec((H, H)),
            _full_spec((1, H)),
            _full_spec((H, H)),
            _full_spec((1, H)),
        ],
        out_specs=pl.BlockSpec((BM, H),
                               lambda s: (jnp.maximum(s - NB, 0), 0)),
        out_shape=jax.ShapeDtypeStruct((N, H), f32),
        scratch_shapes=[
            pltpu.VMEM((N, H), f32),
            pltpu.VMEM((N, H), f32),
        ],
    )(x, A_pos, A_neg, W_in, b_in.reshape(1, H),
      W_pos0, b_pos0.reshape(1, H), W_neg0, b_neg0.reshape(1, H),
      W_pos1, b_pos1.reshape(1, H), W_neg1, b_neg1.reshape(1, H))


# speedup vs baseline: 1.1525x; 1.0136x over previous
import jax
import jax.numpy as jnp
from jax.experimental import pallas as pl
from jax.experimental.pallas import tpu as pltpu

N = 4096
H = 256
BM = 256
NB = N // BM

_DNT = (((1,), (1,)), ((), ()))


def _gcn_kernel(x_ref, Ap_ref, An_ref, Win_ref, bin_ref,
                Wp0_ref, bp0_ref, Wn0_ref, bn0_ref,
                Wp1_ref, bp1_ref, Wn1_ref, bn1_ref,
                out_ref, h0_ref, h1_ref):
    s = pl.program_id(0)

    @pl.when(s == 0)
    def _prep():
        h0_ref[...] = jnp.tanh(
            jax.lax.dot_general(x_ref[...], Win_ref[...], _DNT,
                                preferred_element_type=jnp.float32)
            + bin_ref[...]
        )

    def stripe(h, Wp, bp, Wn, bn):
        hp = jnp.dot(Ap_ref[...], h, preferred_element_type=jnp.float32)
        hn = jnp.dot(An_ref[...], h, preferred_element_type=jnp.float32)
        return jnp.tanh(
            jax.lax.dot_general(hp, Wp, _DNT,
                                preferred_element_type=jnp.float32)
            + bp
            + jax.lax.dot_general(hn, Wn, _DNT,
                                  preferred_element_type=jnp.float32)
            + bn
        )

    @pl.when(s < NB)
    def _layer0():
        t = stripe(h0_ref[...], Wp0_ref[...], bp0_ref[...],
                   Wn0_ref[...], bn0_ref[...])
        h1_ref[pl.ds(s * BM, BM), :] = t

    @pl.when(s >= NB)
    def _layer1():
        out_ref[...] = stripe(h1_ref[...], Wp1_ref[...], bp1_ref[...],
                              Wn1_ref[...], bn1_ref[...])


def _stripe_spec():
    return pl.BlockSpec((BM, N), lambda s: (s % NB, 0))


def _full_spec(shape):
    return pl.BlockSpec(shape, lambda s: (0,) * len(shape))


@jax.jit
def kernel(x, A_pos, A_neg, W_in, b_in, W_pos0, b_pos0, W_neg0, b_neg0,
           W_pos1, b_pos1, W_neg1, b_neg1):
    f32 = jnp.float32
    return pl.pallas_call(
        _gcn_kernel,
        grid=(2 * NB,),
        in_specs=[
            _full_spec((N, H)),
            _stripe_spec(),
            _stripe_spec(),
            _full_spec((H, H)),
            _full_spec((1, H)),
            _full_spec((H, H)),
            _full_spec((1, H)),
            _full_spec((H, H)),
            _full_spec((1, H)),
            _full_spec((H, H)),
            _full_spec((1, H)),
            _full_spec((H, H)),
            _full_spec((1, H)),
        ],
        out_specs=pl.BlockSpec((BM, H),
                               lambda s: (jnp.maximum(s - NB, 0), 0)),
        out_shape=jax.ShapeDtypeStruct((N, H), f32),
        scratch_shapes=[
            pltpu.VMEM((N, H), f32),
            pltpu.VMEM((N, H), f32),
        ],
    )(x, A_pos, A_neg, W_in, b_in.reshape(1, H),
      W_pos0, b_pos0.reshape(1, H), W_neg0, b_neg0.reshape(1, H),
      W_pos1, b_pos1.reshape(1, H), W_neg1, b_neg1.reshape(1, H))
